# Initial kernel scaffold; baseline (speedup 1.0000x reference)
import jax
import jax.numpy as jnp
from jax.experimental import pallas as pl


def kernel(h, w_qc, w_qu, kv_wa, kv_wb, kv_za, kv_zb, kv_ba, kv_bb, w_k, w_v, ix_wa, ix_wb, ix_za, ix_zb, ix_ba, ix_bb, w_dq, w_iuq, w_w, q_scale, k_scale, sink, gp0, gp1, gp2, gp3, w_final):
    raise NotImplementedError("write your pallas kernel here")


# speedup vs baseline: 1.8896x; 1.8896x over previous
import jax
import jax.numpy as jnp
from jax.experimental import pallas as pl

D = 2048
QC = 768
H = 16
DH = 64
ROPE = 32
HALF = ROPE // 2
THETA = 10000.0
M = 16
CI = 64
NIH = 4
TOPK = 16
NG = 4
DG = 512

TQ = 256
NEG = -1e30


def _rope2d(x, cos, sin):
    x1 = x[:, :HALF]
    x2 = x[:, HALF:ROPE]
    xp = x[:, ROPE:]
    return jnp.concatenate([x1 * cos - x2 * sin, x1 * sin + x2 * cos, xp], axis=-1)


def _dot(a, b):
    return jnp.dot(a, b, preferred_element_type=jnp.float32)


def _dot_nt(a, b):
    return jax.lax.dot_general(a, b, (((1,), (1,)), ((), ())),
                               preferred_element_type=jnp.float32)


def _compress_tile(hh, wb, zb, bb, c):
    cb = _dot(hh, wb).reshape(TQ // M, M, c)
    z = _dot(hh, zb).reshape(TQ // M, M, c) + bb[None]
    zmax = jnp.max(z, axis=1, keepdims=True)
    e = jnp.exp(z - zmax)
    w = e / jnp.sum(e, axis=1, keepdims=True)
    return jnp.sum(w * cb, axis=1)


def _proj_kernel(h_ref, wqc_ref, wqu_ref, wk_ref, wv_ref,
                 kvwb_ref, kvzb_ref, kvbb_ref, ixwb_ref, ixzb_ref, ixbb_ref,
                 kscale_ref, cos_ref, sin_ref,
                 q_ref, k_ref, v_ref, kc_ref, kci_ref):
    hh = h_ref[...]
    q_ref[...] = _dot(_dot(hh, wqc_ref[...]), wqu_ref[...])
    k = _dot(hh, wk_ref[...])
    k = k * jax.lax.rsqrt(jnp.mean(k * k, axis=-1, keepdims=True) + 1e-6)
    k = k * kscale_ref[...]
    k_ref[...] = _rope2d(k, cos_ref[...], sin_ref[...])
    v_ref[...] = _dot(hh, wv_ref[...])
    kc_ref[...] = _compress_tile(hh, kvwb_ref[...], kvzb_ref[...],
                                 kvbb_ref[...], DH)
    kci_ref[...] = _compress_tile(hh, ixwb_ref[...], ixzb_ref[...],
                                  ixbb_ref[...], CI)


def _indexer_kernel(h_ref, wdq_ref, wiuq_ref, ww_ref, kci_ref, sel_ref, *, nb):
    i = pl.program_id(0)
    hh = h_ref[...]
    cq = _dot(hh, wdq_ref[...])
    qi = _dot(cq, wiuq_ref[...])
    wtok = _dot(hh, ww_ref[...])
    kci = kci_ref[...]
    scores = jnp.zeros((TQ, nb), jnp.float32)
    for hix in range(NIH):
        s = _dot_nt(qi[:, hix * CI:(hix + 1) * CI], kci)
        scores = scores + wtok[:, hix:hix + 1] * jnp.maximum(s, 0.0)
    qidx = i * TQ + jax.lax.broadcasted_iota(jnp.int32, (TQ, nb), 0)
    colid = jax.lax.broadcasted_iota(jnp.int32, (TQ, nb), 1)
    bend = colid * M + (M - 1)
    scores = jnp.where(bend < qidx, scores, -jnp.inf)
    sel = jnp.zeros((TQ, nb), jnp.float32)
    for _ in range(TOPK):
        m = jnp.max(scores, axis=1, keepdims=True)
        idx = jnp.min(jnp.where(scores == m, colid, nb), axis=1, keepdims=True)
        chosen = (colid == idx) & (m > -jnp.inf)
        sel = jnp.where(chosen, 1.0, sel)
        scores = jnp.where(chosen, -jnp.inf, scores)
    sel_ref[...] = sel


def _attn_kernel(q_ref, k_ref, v_ref, kc_ref, sel_ref, cos_ref, sin_ref,
                 qscale_ref, sink_ref, o_ref, *, t, nb):
    i = pl.program_id(0)
    k = k_ref[...]
    v = v_ref[...]
    kc = kc_ref[...]
    sel = sel_ref[...]
    erow = jax.lax.broadcasted_iota(jnp.int32, (nb, t), 0)
    ecol = jax.lax.broadcasted_iota(jnp.int32, (nb, t), 1)
    expand = ((ecol // M) == erow).astype(jnp.float32)
    mask_f = _dot(sel, expand)
    qidx = i * TQ + jax.lax.broadcasted_iota(jnp.int32, (TQ, nb), 0)
    bend = jax.lax.broadcasted_iota(jnp.int32, (TQ, nb), 1) * M + (M - 1)
    cmask = bend < qidx
    cos = cos_ref[...]
    sin = sin_ref[...]
    qsc = qscale_ref[...]
    sink = sink_ref[...]
    scale = 1.0 / jnp.sqrt(jnp.float32(DH))
    for h in range(H):
        qh = q_ref[:, h * DH:(h + 1) * DH]
        qh = qh * jax.lax.rsqrt(jnp.mean(qh * qh, axis=-1, keepdims=True) + 1e-6)
        qh = _rope2d(qh * qsc, cos, sin)
        s_f = _dot_nt(qh, k) * scale
        s_f = jnp.where(mask_f > 0.5, s_f, NEG)
        s_c = _dot_nt(qh, kc) * scale
        s_c = jnp.where(cmask, s_c, NEG)
        snk = sink[0:1, h:h + 1]
        m = jnp.maximum(jnp.max(s_f, axis=1, keepdims=True),
                        jnp.max(s_c, axis=1, keepdims=True))
        m = jnp.maximum(m, snk)
        ef = jnp.exp(s_f - m)
        ec = jnp.exp(s_c - m)
        es = jnp.exp(snk - m)
        den = (jnp.sum(ef, axis=1, keepdims=True)
               + jnp.sum(ec, axis=1, keepdims=True) + es)
        o_ref[:, h * DH:(h + 1) * DH] = (_dot(ec, kc) + _dot(ef, v)) / den


def _outproj_kernel(o_ref, gp0_ref, gp1_ref, gp2_ref, gp3_ref, wf_ref, out_ref):
    o = o_ref[...]
    hp = (H // NG) * DH
    proj = jnp.concatenate([
        _dot(o[:, 0 * hp:1 * hp], gp0_ref[...]),
        _dot(o[:, 1 * hp:2 * hp], gp1_ref[...]),
        _dot(o[:, 2 * hp:3 * hp], gp2_ref[...]),
        _dot(o[:, 3 * hp:4 * hp], gp3_ref[...]),
    ], axis=1)
    out_ref[...] = _dot(proj, wf_ref[...])


def _full(a):
    return pl.BlockSpec(a.shape, lambda i: (0,) * a.ndim)


def _rowtile(shape):
    return pl.BlockSpec((TQ,) + shape[1:], lambda i: (i,) + (0,) * (len(shape) - 1))


def kernel(h, w_qc, w_qu, kv_wa, kv_wb, kv_za, kv_zb, kv_ba, kv_bb, w_k, w_v,
           ix_wa, ix_wb, ix_za, ix_zb, ix_ba, ix_bb, w_dq, w_iuq, w_w,
           q_scale, k_scale, sink, gp0, gp1, gp2, gp3, w_final):
    b, t, _ = h.shape
    nb = t // M
    grid = t // TQ
    h2 = h.reshape(t, D)
    f32 = jnp.float32

    inv = 1.0 / (THETA ** (jnp.arange(0, ROPE, 2, dtype=f32) / ROPE))
    ang = jnp.arange(t, dtype=f32)[:, None] * inv
    cos, sin = jnp.cos(ang), jnp.sin(ang)
    k_scale2 = k_scale.reshape(1, DH)
    q_scale2 = q_scale.reshape(1, DH)
    sink2 = sink.reshape(1, H)

    q_raw, k_rot, v_p, kcomp, kcomp_i = pl.pallas_call(
        _proj_kernel,
        grid=(grid,),
        in_specs=[
            _rowtile(h2.shape),
            _full(w_qc), _full(w_qu), _full(w_k), _full(w_v),
            _full(kv_wb), _full(kv_zb), _full(kv_bb),
            _full(ix_wb), _full(ix_zb), _full(ix_bb),
            _full(k_scale2),
            _rowtile(cos.shape), _rowtile(sin.shape),
        ],
        out_specs=[
            _rowtile((t, H * DH)),
            _rowtile((t, DH)),
            _rowtile((t, DH)),
            pl.BlockSpec((TQ // M, DH), lambda i: (i, 0)),
            pl.BlockSpec((TQ // M, CI), lambda i: (i, 0)),
        ],
        out_shape=[
            jax.ShapeDtypeStruct((t, H * DH), f32),
            jax.ShapeDtypeStruct((t, DH), f32),
            jax.ShapeDtypeStruct((t, DH), f32),
            jax.ShapeDtypeStruct((nb, DH), f32),
            jax.ShapeDtypeStruct((nb, CI), f32),
        ],
    )(h2, w_qc, w_qu, w_k, w_v, kv_wb, kv_zb, kv_bb, ix_wb, ix_zb, ix_bb,
      k_scale2, cos, sin)

    import functools
    sel = pl.pallas_call(
        functools.partial(_indexer_kernel, nb=nb),
        grid=(grid,),
        in_specs=[
            _rowtile(h2.shape),
            _full(w_dq), _full(w_iuq), _full(w_w), _full(kcomp_i),
        ],
        out_specs=_rowtile((t, nb)),
        out_shape=jax.ShapeDtypeStruct((t, nb), f32),
    )(h2, w_dq, w_iuq, w_w, kcomp_i)

    o = pl.pallas_call(
        functools.partial(_attn_kernel, t=t, nb=nb),
        grid=(grid,),
        in_specs=[
            _rowtile(q_raw.shape),
            _full(k_rot), _full(v_p), _full(kcomp),
            _rowtile(sel.shape),
            _rowtile(cos.shape), _rowtile(sin.shape),
            _full(q_scale2), _full(sink2),
        ],
        out_specs=_rowtile((t, H * DH)),
        out_shape=jax.ShapeDtypeStruct((t, H * DH), f32),
    )(q_raw, k_rot, v_p, kcomp, sel, cos, sin, q_scale2, sink2)

    out = pl.pallas_call(
        _outproj_kernel,
        grid=(grid,),
        in_specs=[
            _rowtile(o.shape),
            _full(gp0), _full(gp1), _full(gp2), _full(gp3), _full(w_final),
        ],
        out_specs=_rowtile((t, D)),
        out_shape=jax.ShapeDtypeStruct((t, D), f32),
    )(o, gp0, gp1, gp2, gp3, w_final)

    return out.reshape(b, t, D)


# speedup vs baseline: 2.5878x; 1.3695x over previous
import jax
import jax.numpy as jnp
from jax.experimental import pallas as pl

D = 2048
QC = 768
H = 16
DH = 64
ROPE = 32
HALF = ROPE // 2
THETA = 10000.0
M = 16
CI = 64
NIH = 4
TOPK = 16
NG = 4
DG = 512

TQ = 256
NEG = -1e30


def _rope2d(x, cos, sin):
    x1 = x[:, :HALF]
    x2 = x[:, HALF:ROPE]
    xp = x[:, ROPE:]
    return jnp.concatenate([x1 * cos - x2 * sin, x1 * sin + x2 * cos, xp], axis=-1)


def _dot(a, b):
    return jnp.dot(a, b, preferred_element_type=jnp.float32)


def _dot_nt(a, b):
    return jax.lax.dot_general(a, b, (((1,), (1,)), ((), ())),
                               preferred_element_type=jnp.float32)


def _compress_tile(hh, wb, zb, bb, c):
    cb = _dot(hh, wb).reshape(TQ // M, M, c)
    z = _dot(hh, zb).reshape(TQ // M, M, c) + bb[None]
    zmax = jnp.max(z, axis=1, keepdims=True)
    e = jnp.exp(z - zmax)
    w = e / jnp.sum(e, axis=1, keepdims=True)
    return jnp.sum(w * cb, axis=1)


def _proj_kernel(h_ref, wqc_ref, wqu_ref, wk_ref, wv_ref,
                 kvwb_ref, kvzb_ref, kvbb_ref, ixwb_ref, ixzb_ref, ixbb_ref,
                 kscale_ref, cos_ref, sin_ref,
                 q_ref, k_ref, vaug_ref, kcaug_ref, kci_ref):
    hh = h_ref[...]
    q_ref[...] = _dot(_dot(hh, wqc_ref[...]), wqu_ref[...])
    k = _dot(hh, wk_ref[...])
    k = k * jax.lax.rsqrt(jnp.mean(k * k, axis=-1, keepdims=True) + 1e-6)
    k = k * kscale_ref[...]
    k_ref[...] = _rope2d(k, cos_ref[...], sin_ref[...])
    v = _dot(hh, wv_ref[...])
    vaug_ref[...] = jnp.concatenate(
        [v, jnp.ones((TQ, 1), jnp.float32)], axis=1)
    kc = _compress_tile(hh, kvwb_ref[...], kvzb_ref[...], kvbb_ref[...], DH)
    kcaug_ref[...] = jnp.concatenate(
        [kc, jnp.ones((TQ // M, 1), jnp.float32)], axis=1)
    kci_ref[...] = _compress_tile(hh, ixwb_ref[...], ixzb_ref[...],
                                  ixbb_ref[...], CI)


def _indexer_kernel(h_ref, wdq_ref, wiuq_ref, ww_ref, kci_ref, sel_ref, *, nb):
    i = pl.program_id(0)
    hh = h_ref[...]
    cq = _dot(hh, wdq_ref[...])
    qi = _dot(cq, wiuq_ref[...])
    wtok = _dot(hh, ww_ref[...])
    kci = kci_ref[...]
    scores = jnp.zeros((TQ, nb), jnp.float32)
    for hix in range(NIH):
        s = _dot_nt(qi[:, hix * CI:(hix + 1) * CI], kci)
        scores = scores + wtok[:, hix:hix + 1] * jnp.maximum(s, 0.0)
    qidx = i * TQ + jax.lax.broadcasted_iota(jnp.int32, (TQ, nb), 0)
    colid = jax.lax.broadcasted_iota(jnp.int32, (TQ, nb), 1)
    bend = colid * M + (M - 1)
    scores = jnp.where(bend < qidx, scores, -jnp.inf)
    sel = jnp.zeros((TQ, nb), jnp.float32)
    for _ in range(TOPK):
        m = jnp.max(scores, axis=1, keepdims=True)
        idx = jnp.min(jnp.where(scores == m, colid, nb), axis=1, keepdims=True)
        chosen = (colid == idx) & (m > -jnp.inf)
        sel = jnp.where(chosen, 1.0, sel)
        scores = jnp.where(chosen, -jnp.inf, scores)
    sel_ref[...] = sel


def _attn_kernel(q_ref, k_ref, vaug_ref, kcaug_ref, sel_ref,
                 cos_ref, sin_ref, qscale_ref, sink_ref, o_ref, *, t, nb):
    i = pl.program_id(0)
    k = k_ref[...]
    vaug = vaug_ref[...]
    kcaug = kcaug_ref[...]
    kc = kcaug[:, :DH]
    sel = sel_ref[...]
    erow = jax.lax.broadcasted_iota(jnp.int32, (nb, t), 0)
    ecol = jax.lax.broadcasted_iota(jnp.int32, (nb, t), 1)
    expand = ((ecol // M) == erow).astype(jnp.float32)
    mask_f = _dot(sel, expand)
    qidx = i * TQ + jax.lax.broadcasted_iota(jnp.int32, (TQ, nb), 0)
    bend = jax.lax.broadcasted_iota(jnp.int32, (TQ, nb), 1) * M + (M - 1)
    cmask = (bend < qidx).astype(jnp.float32)
    cos = cos_ref[...]
    sin = sin_ref[...]
    qsc = qscale_ref[...] * (1.0 / jnp.sqrt(jnp.float32(DH)))
    sink = sink_ref[...]
    for h in range(H):
        qh = q_ref[:, h * DH:(h + 1) * DH]
        qh = qh * jax.lax.rsqrt(jnp.mean(qh * qh, axis=-1, keepdims=True) + 1e-6)
        qh = _rope2d(qh * qsc, cos, sin)
        ef = jnp.exp(_dot_nt(qh, k)) * mask_f
        ec = jnp.exp(_dot_nt(qh, kc)) * cmask
        es = jnp.exp(sink[0:1, h:h + 1])
        acc = _dot(ec, kcaug) + _dot(ef, vaug)
        o_ref[:, h * DH:(h + 1) * DH] = (acc[:, :DH]
                                         / (acc[:, DH:DH + 1] + es))


def _outproj_kernel(o_ref, gp0_ref, gp1_ref, gp2_ref, gp3_ref, wf_ref, out_ref):
    o = o_ref[...]
    hp = (H // NG) * DH
    proj = jnp.concatenate([
        _dot(o[:, 0 * hp:1 * hp], gp0_ref[...]),
        _dot(o[:, 1 * hp:2 * hp], gp1_ref[...]),
        _dot(o[:, 2 * hp:3 * hp], gp2_ref[...]),
        _dot(o[:, 3 * hp:4 * hp], gp3_ref[...]),
    ], axis=1)
    out_ref[...] = _dot(proj, wf_ref[...])


def _full(a):
    return pl.BlockSpec(a.shape, lambda i: (0,) * a.ndim)


def _rowtile(shape):
    return pl.BlockSpec((TQ,) + shape[1:], lambda i: (i,) + (0,) * (len(shape) - 1))


def kernel(h, w_qc, w_qu, kv_wa, kv_wb, kv_za, kv_zb, kv_ba, kv_bb, w_k, w_v,
           ix_wa, ix_wb, ix_za, ix_zb, ix_ba, ix_bb, w_dq, w_iuq, w_w,
           q_scale, k_scale, sink, gp0, gp1, gp2, gp3, w_final):
    b, t, _ = h.shape
    nb = t // M
    grid = t // TQ
    h2 = h.reshape(t, D)
    f32 = jnp.float32

    inv = 1.0 / (THETA ** (jnp.arange(0, ROPE, 2, dtype=f32) / ROPE))
    ang = jnp.arange(t, dtype=f32)[:, None] * inv
    cos, sin = jnp.cos(ang), jnp.sin(ang)
    k_scale2 = k_scale.reshape(1, DH)
    q_scale2 = q_scale.reshape(1, DH)
    sink2 = sink.reshape(1, H)

    q_raw, k_rot, v_aug, kc_aug, kcomp_i = pl.pallas_call(
        _proj_kernel,
        grid=(grid,),
        in_specs=[
            _rowtile(h2.shape),
            _full(w_qc), _full(w_qu), _full(w_k), _full(w_v),
            _full(kv_wb), _full(kv_zb), _full(kv_bb),
            _full(ix_wb), _full(ix_zb), _full(ix_bb),
            _full(k_scale2),
            _rowtile(cos.shape), _rowtile(sin.shape),
        ],
        out_specs=[
            _rowtile((t, H * DH)),
            _rowtile((t, DH)),
            _rowtile((t, DH + 1)),
            pl.BlockSpec((TQ // M, DH + 1), lambda i: (i, 0)),
            pl.BlockSpec((TQ // M, CI), lambda i: (i, 0)),
        ],
        out_shape=[
            jax.ShapeDtypeStruct((t, H * DH), f32),
            jax.ShapeDtypeStruct((t, DH), f32),
            jax.ShapeDtypeStruct((t, DH + 1), f32),
            jax.ShapeDtypeStruct((nb, DH + 1), f32),
            jax.ShapeDtypeStruct((nb, CI), f32),
        ],
    )(h2, w_qc, w_qu, w_k, w_v, kv_wb, kv_zb, kv_bb, ix_wb, ix_zb, ix_bb,
      k_scale2, cos, sin)

    import functools
    sel = pl.pallas_call(
        functools.partial(_indexer_kernel, nb=nb),
        grid=(grid,),
        in_specs=[
            _rowtile(h2.shape),
            _full(w_dq), _full(w_iuq), _full(w_w), _full(kcomp_i),
        ],
        out_specs=_rowtile((t, nb)),
        out_shape=jax.ShapeDtypeStruct((t, nb), f32),
    )(h2, w_dq, w_iuq, w_w, kcomp_i)

    o = pl.pallas_call(
        functools.partial(_attn_kernel, t=t, nb=nb),
        grid=(grid,),
        in_specs=[
            _rowtile(q_raw.shape),
            _full(k_rot), _full(v_aug), _full(kc_aug),
            _rowtile(sel.shape),
            _rowtile(cos.shape), _rowtile(sin.shape),
            _full(q_scale2), _full(sink2),
        ],
        out_specs=_rowtile((t, H * DH)),
        out_shape=jax.ShapeDtypeStruct((t, H * DH), f32),
    )(q_raw, k_rot, v_aug, kc_aug, sel, cos, sin, q_scale2, sink2)

    out = pl.pallas_call(
        _outproj_kernel,
        grid=(grid,),
        in_specs=[
            _rowtile(o.shape),
            _full(gp0), _full(gp1), _full(gp2), _full(gp3), _full(w_final),
        ],
        out_specs=_rowtile((t, D)),
        out_shape=jax.ShapeDtypeStruct((t, D), f32),
    )(o, gp0, gp1, gp2, gp3, w_final)

    return out.reshape(b, t, D)


# speedup vs baseline: 2.7150x; 1.0491x over previous
import jax
import jax.numpy as jnp
from jax.experimental import pallas as pl

D = 2048
QC = 768
H = 16
DH = 64
ROPE = 32
HALF = ROPE // 2
THETA = 10000.0
M = 16
CI = 64
NIH = 4
TOPK = 16
NG = 4
DG = 512

TQ = 256
NEG = -1e30


def _rope2d(x, cos, sin):
    x1 = x[:, :HALF]
    x2 = x[:, HALF:ROPE]
    xp = x[:, ROPE:]
    return jnp.concatenate([x1 * cos - x2 * sin, x1 * sin + x2 * cos, xp], axis=-1)


def _dot(a, b):
    return jnp.dot(a, b, preferred_element_type=jnp.float32)


def _dot_nt(a, b):
    return jax.lax.dot_general(a, b, (((1,), (1,)), ((), ())),
                               preferred_element_type=jnp.float32)


def _compress_tile(hh, wb, zb, bb, c):
    cb = _dot(hh, wb).reshape(TQ // M, M, c)
    z = _dot(hh, zb).reshape(TQ // M, M, c) + bb[None]
    zmax = jnp.max(z, axis=1, keepdims=True)
    e = jnp.exp(z - zmax)
    w = e / jnp.sum(e, axis=1, keepdims=True)
    return jnp.sum(w * cb, axis=1)


def _proj_kernel(h_ref, wqc_ref, wqu_ref, wk_ref, wv_ref,
                 kvwb_ref, kvzb_ref, kvbb_ref, ixwb_ref, ixzb_ref, ixbb_ref,
                 wdq_ref, wiuq_ref, ww_ref,
                 kscale_ref, cos_ref, sin_ref,
                 q_ref, k_ref, vaug_ref, kcaug_ref, kci_ref, qi_ref, wtok_ref):
    hh = h_ref[...]
    qi_ref[...] = _dot(_dot(hh, wdq_ref[...]), wiuq_ref[...])
    wtok_ref[...] = _dot(hh, ww_ref[...])
    q_ref[...] = _dot(_dot(hh, wqc_ref[...]), wqu_ref[...])
    k = _dot(hh, wk_ref[...])
    k = k * jax.lax.rsqrt(jnp.mean(k * k, axis=-1, keepdims=True) + 1e-6)
    k = k * kscale_ref[...]
    k_ref[...] = _rope2d(k, cos_ref[...], sin_ref[...])
    v = _dot(hh, wv_ref[...])
    vaug_ref[...] = jnp.concatenate(
        [v, jnp.ones((TQ, 1), jnp.float32)], axis=1)
    kc = _compress_tile(hh, kvwb_ref[...], kvzb_ref[...], kvbb_ref[...], DH)
    kcaug_ref[...] = jnp.concatenate(
        [kc, jnp.ones((TQ // M, 1), jnp.float32)], axis=1)
    kci_ref[...] = _compress_tile(hh, ixwb_ref[...], ixzb_ref[...],
                                  ixbb_ref[...], CI)


def _indexer_kernel(qi_ref, wtok_ref, kci_ref, sel_ref, *, nb):
    i = pl.program_id(0)
    qi = qi_ref[...]
    wtok = wtok_ref[...]
    kci = kci_ref[...]
    scores = jnp.zeros((TQ, nb), jnp.float32)
    for hix in range(NIH):
        s = _dot_nt(qi[:, hix * CI:(hix + 1) * CI], kci)
        scores = scores + wtok[:, hix:hix + 1] * jnp.maximum(s, 0.0)
    qidx = i * TQ + jax.lax.broadcasted_iota(jnp.int32, (TQ, nb), 0)
    colid = jax.lax.broadcasted_iota(jnp.int32, (TQ, nb), 1)
    bend = colid * M + (M - 1)
    scores = jnp.where(bend < qidx, scores, -jnp.inf)
    sel = jnp.zeros((TQ, nb), jnp.float32)
    for _ in range(TOPK):
        m = jnp.max(scores, axis=1, keepdims=True)
        idx = jnp.min(jnp.where(scores == m, colid, nb), axis=1, keepdims=True)
        chosen = (colid == idx) & (m > -jnp.inf)
        sel = jnp.where(chosen, 1.0, sel)
        scores = jnp.where(chosen, -jnp.inf, scores)
    sel_ref[...] = sel


def _attn_kernel(q_ref, k_ref, vaug_ref, kcaug_ref, sel_ref,
                 cos_ref, sin_ref, qscale_ref, sink_ref,
                 gp0_ref, gp1_ref, gp2_ref, gp3_ref, wf_ref,
                 out_ref, *, t, nb):
    i = pl.program_id(0)
    k = k_ref[...]
    vaug = vaug_ref[...]
    kcaug = kcaug_ref[...]
    kc = kcaug[:, :DH]
    sel = sel_ref[...]
    erow = jax.lax.broadcasted_iota(jnp.int32, (nb, t), 0)
    ecol = jax.lax.broadcasted_iota(jnp.int32, (nb, t), 1)
    expand = ((ecol // M) == erow).astype(jnp.float32)
    mask_f = _dot(sel, expand)
    qidx = i * TQ + jax.lax.broadcasted_iota(jnp.int32, (TQ, nb), 0)
    bend = jax.lax.broadcasted_iota(jnp.int32, (TQ, nb), 1) * M + (M - 1)
    cmask = (bend < qidx).astype(jnp.float32)
    cos = cos_ref[...]
    sin = sin_ref[...]
    qsc = qscale_ref[...] * (1.0 / jnp.sqrt(jnp.float32(DH)))
    sink = sink_ref[...]
    heads = []
    for h in range(H):
        qh = q_ref[:, h * DH:(h + 1) * DH]
        qh = qh * jax.lax.rsqrt(jnp.mean(qh * qh, axis=-1, keepdims=True) + 1e-6)
        qh = _rope2d(qh * qsc, cos, sin)
        ef = jnp.exp(_dot_nt(qh, k)) * mask_f
        ec = jnp.exp(_dot_nt(qh, kc)) * cmask
        es = jnp.exp(sink[0:1, h:h + 1])
        acc = _dot(ec, kcaug) + _dot(ef, vaug)
        heads.append(acc[:, :DH] / (acc[:, DH:DH + 1] + es))
    hp_n = H // NG
    gps = (gp0_ref, gp1_ref, gp2_ref, gp3_ref)
    proj = jnp.concatenate([
        _dot(jnp.concatenate(heads[g * hp_n:(g + 1) * hp_n], axis=1),
             gps[g][...])
        for g in range(NG)
    ], axis=1)
    out_ref[...] = _dot(proj, wf_ref[...])


def _full(a):
    return pl.BlockSpec(a.shape, lambda i: (0,) * a.ndim)


def _rowtile(shape):
    return pl.BlockSpec((TQ,) + shape[1:], lambda i: (i,) + (0,) * (len(shape) - 1))


def kernel(h, w_qc, w_qu, kv_wa, kv_wb, kv_za, kv_zb, kv_ba, kv_bb, w_k, w_v,
           ix_wa, ix_wb, ix_za, ix_zb, ix_ba, ix_bb, w_dq, w_iuq, w_w,
           q_scale, k_scale, sink, gp0, gp1, gp2, gp3, w_final):
    b, t, _ = h.shape
    nb = t // M
    grid = t // TQ
    h2 = h.reshape(t, D)
    f32 = jnp.float32

    inv = 1.0 / (THETA ** (jnp.arange(0, ROPE, 2, dtype=f32) / ROPE))
    ang = jnp.arange(t, dtype=f32)[:, None] * inv
    cos, sin = jnp.cos(ang), jnp.sin(ang)
    k_scale2 = k_scale.reshape(1, DH)
    q_scale2 = q_scale.reshape(1, DH)
    sink2 = sink.reshape(1, H)

    q_raw, k_rot, v_aug, kc_aug, kcomp_i, qi_p, wtok_p = pl.pallas_call(
        _proj_kernel,
        grid=(grid,),
        in_specs=[
            _rowtile(h2.shape),
            _full(w_qc), _full(w_qu), _full(w_k), _full(w_v),
            _full(kv_wb), _full(kv_zb), _full(kv_bb),
            _full(ix_wb), _full(ix_zb), _full(ix_bb),
            _full(w_dq), _full(w_iuq), _full(w_w),
            _full(k_scale2),
            _rowtile(cos.shape), _rowtile(sin.shape),
        ],
        out_specs=[
            _rowtile((t, H * DH)),
            _rowtile((t, DH)),
            _rowtile((t, DH + 1)),
            pl.BlockSpec((TQ // M, DH + 1), lambda i: (i, 0)),
            pl.BlockSpec((TQ // M, CI), lambda i: (i, 0)),
            _rowtile((t, NIH * CI)),
            _rowtile((t, NIH)),
        ],
        out_shape=[
            jax.ShapeDtypeStruct((t, H * DH), f32),
            jax.ShapeDtypeStruct((t, DH), f32),
            jax.ShapeDtypeStruct((t, DH + 1), f32),
            jax.ShapeDtypeStruct((nb, DH + 1), f32),
            jax.ShapeDtypeStruct((nb, CI), f32),
            jax.ShapeDtypeStruct((t, NIH * CI), f32),
            jax.ShapeDtypeStruct((t, NIH), f32),
        ],
    )(h2, w_qc, w_qu, w_k, w_v, kv_wb, kv_zb, kv_bb, ix_wb, ix_zb, ix_bb,
      w_dq, w_iuq, w_w, k_scale2, cos, sin)

    import functools
    sel = pl.pallas_call(
        functools.partial(_indexer_kernel, nb=nb),
        grid=(grid,),
        in_specs=[
            _rowtile(qi_p.shape),
            _rowtile(wtok_p.shape),
            _full(kcomp_i),
        ],
        out_specs=_rowtile((t, nb)),
        out_shape=jax.ShapeDtypeStruct((t, nb), f32),
    )(qi_p, wtok_p, kcomp_i)

    out = pl.pallas_call(
        functools.partial(_attn_kernel, t=t, nb=nb),
        grid=(grid,),
        in_specs=[
            _rowtile(q_raw.shape),
            _full(k_rot), _full(v_aug), _full(kc_aug),
            _rowtile(sel.shape),
            _rowtile(cos.shape), _rowtile(sin.shape),
            _full(q_scale2), _full(sink2),
            _full(gp0), _full(gp1), _full(gp2), _full(gp3), _full(w_final),
        ],
        out_specs=_rowtile((t, D)),
        out_shape=jax.ShapeDtypeStruct((t, D), f32),
    )(q_raw, k_rot, v_aug, kc_aug, sel, cos, sin, q_scale2, sink2,
      gp0, gp1, gp2, gp3, w_final)

    return out.reshape(b, t, D)


# speedup vs baseline: 3.0039x; 1.1064x over previous
import jax
import jax.numpy as jnp
from jax.experimental import pallas as pl

D = 2048
QC = 768
H = 16
DH = 64
ROPE = 32
HALF = ROPE // 2
THETA = 10000.0
M = 16
CI = 64
NIH = 4
TOPK = 16
NG = 4
DG = 512

TQ = 256
NEG = -1e30


def _rope2d(x, cos, sin):
    x1 = x[:, :HALF]
    x2 = x[:, HALF:ROPE]
    xp = x[:, ROPE:]
    return jnp.concatenate([x1 * cos - x2 * sin, x1 * sin + x2 * cos, xp], axis=-1)


def _dot(a, b):
    return jnp.dot(a, b, preferred_element_type=jnp.float32)


def _dot_nt(a, b):
    return jax.lax.dot_general(a, b, (((1,), (1,)), ((), ())),
                               preferred_element_type=jnp.float32)


def _compress_tile(hh, wb, zb, bb, c):
    cb = _dot(hh, wb).reshape(TQ // M, M, c)
    z = _dot(hh, zb).reshape(TQ // M, M, c) + bb[None]
    zmax = jnp.max(z, axis=1, keepdims=True)
    e = jnp.exp(z - zmax)
    w = e / jnp.sum(e, axis=1, keepdims=True)
    return jnp.sum(w * cb, axis=1)


def _proj_kernel(h_ref, wqc_ref, wqu_ref, wk_ref, wv_ref,
                 kvwb_ref, kvzb_ref, kvbb_ref, ixwb_ref, ixzb_ref, ixbb_ref,
                 wdq_ref, wiuq_ref, ww_ref,
                 kscale_ref, cos_ref, sin_ref,
                 q_ref, k_ref, vaug_ref, kcaug_ref, kci_ref, qi_ref, wtok_ref):
    hh = h_ref[...]
    qi_ref[...] = _dot(_dot(hh, wdq_ref[...]), wiuq_ref[...])
    wtok_ref[...] = _dot(hh, ww_ref[...])
    q_ref[...] = _dot(_dot(hh, wqc_ref[...]), wqu_ref[...])
    k = _dot(hh, wk_ref[...])
    k = k * jax.lax.rsqrt(jnp.mean(k * k, axis=-1, keepdims=True) + 1e-6)
    k = k * kscale_ref[...]
    k_ref[...] = _rope2d(k, cos_ref[...], sin_ref[...])
    v = _dot(hh, wv_ref[...])
    vaug_ref[...] = jnp.concatenate(
        [v, jnp.ones((TQ, 1), jnp.float32)], axis=1)
    kc = _compress_tile(hh, kvwb_ref[...], kvzb_ref[...], kvbb_ref[...], DH)
    kcaug_ref[...] = jnp.concatenate(
        [kc, jnp.ones((TQ // M, 1), jnp.float32)], axis=1)
    kci_ref[...] = _compress_tile(hh, ixwb_ref[...], ixzb_ref[...],
                                  ixbb_ref[...], CI)


def _indexer_kernel(qi_ref, wtok_ref, kci_ref, sel_ref, *, nb):
    rows = sel_ref.shape[0]
    qi = qi_ref[...]
    wtok = wtok_ref[...]
    kci = kci_ref[...]
    scores = jnp.zeros((rows, nb), jnp.float32)
    for hix in range(NIH):
        s = _dot_nt(qi[:, hix * CI:(hix + 1) * CI], kci)
        scores = scores + wtok[:, hix:hix + 1] * jnp.maximum(s, 0.0)
    qidx = jax.lax.broadcasted_iota(jnp.int32, (rows, nb), 0)
    colid = jax.lax.broadcasted_iota(jnp.int32, (rows, nb), 1)
    bend = colid * M + (M - 1)
    scores = jnp.where(bend < qidx, scores, -jnp.inf)
    colf = colid.astype(jnp.float32)
    nbf = jnp.float32(nb)
    sel = jnp.zeros((rows, nb), jnp.float32)
    for _ in range(TOPK):
        m = jnp.max(scores, axis=1, keepdims=True)
        idx = jnp.min(jnp.where(scores == m, colf, nbf), axis=1, keepdims=True)
        chosen = (colf == idx) & (m > -jnp.inf)
        sel = jnp.where(chosen, 1.0, sel)
        scores = jnp.where(chosen, -jnp.inf, scores)
    sel_ref[...] = sel


def _attn_kernel(q_ref, k_ref, vaug_ref, kcaug_ref, sel_ref,
                 cos_ref, sin_ref, qscale_ref, sink_ref,
                 gp0_ref, gp1_ref, gp2_ref, gp3_ref, wf_ref,
                 out_ref, *, t, nb):
    i = pl.program_id(0)
    k = k_ref[...]
    vaug = vaug_ref[...]
    kcaug = kcaug_ref[...]
    kc = kcaug[:, :DH]
    sel = sel_ref[...]
    erow = jax.lax.broadcasted_iota(jnp.int32, (nb, t), 0)
    ecol = jax.lax.broadcasted_iota(jnp.int32, (nb, t), 1)
    expand = ((ecol // M) == erow).astype(jnp.float32)
    mask_f = _dot(sel, expand)
    qidx = i * TQ + jax.lax.broadcasted_iota(jnp.int32, (TQ, nb), 0)
    bend = jax.lax.broadcasted_iota(jnp.int32, (TQ, nb), 1) * M + (M - 1)
    cmask = (bend < qidx).astype(jnp.float32)
    cos = cos_ref[...]
    sin = sin_ref[...]
    qsc = qscale_ref[...] * (1.0 / jnp.sqrt(jnp.float32(DH)))
    sink = sink_ref[...]
    heads = []
    for h in range(H):
        qh = q_ref[:, h * DH:(h + 1) * DH]
        qh = qh * jax.lax.rsqrt(jnp.mean(qh * qh, axis=-1, keepdims=True) + 1e-6)
        qh = _rope2d(qh * qsc, cos, sin)
        ef = jnp.exp(_dot_nt(qh, k)) * mask_f
        ec = jnp.exp(_dot_nt(qh, kc)) * cmask
        es = jnp.exp(sink[0:1, h:h + 1])
        acc = _dot(ec, kcaug) + _dot(ef, vaug)
        heads.append(acc[:, :DH] / (acc[:, DH:DH + 1] + es))
    hp_n = H // NG
    gps = (gp0_ref, gp1_ref, gp2_ref, gp3_ref)
    proj = jnp.concatenate([
        _dot(jnp.concatenate(heads[g * hp_n:(g + 1) * hp_n], axis=1),
             gps[g][...])
        for g in range(NG)
    ], axis=1)
    out_ref[...] = _dot(proj, wf_ref[...])


def _full(a):
    return pl.BlockSpec(a.shape, lambda i: (0,) * a.ndim)


def _rowtile(shape):
    return pl.BlockSpec((TQ,) + shape[1:], lambda i: (i,) + (0,) * (len(shape) - 1))


def kernel(h, w_qc, w_qu, kv_wa, kv_wb, kv_za, kv_zb, kv_ba, kv_bb, w_k, w_v,
           ix_wa, ix_wb, ix_za, ix_zb, ix_ba, ix_bb, w_dq, w_iuq, w_w,
           q_scale, k_scale, sink, gp0, gp1, gp2, gp3, w_final):
    b, t, _ = h.shape
    nb = t // M
    grid = t // TQ
    h2 = h.reshape(t, D)
    f32 = jnp.float32

    inv = 1.0 / (THETA ** (jnp.arange(0, ROPE, 2, dtype=f32) / ROPE))
    ang = jnp.arange(t, dtype=f32)[:, None] * inv
    cos, sin = jnp.cos(ang), jnp.sin(ang)
    k_scale2 = k_scale.reshape(1, DH)
    q_scale2 = q_scale.reshape(1, DH)
    sink2 = sink.reshape(1, H)

    q_raw, k_rot, v_aug, kc_aug, kcomp_i, qi_p, wtok_p = pl.pallas_call(
        _proj_kernel,
        grid=(grid,),
        in_specs=[
            _rowtile(h2.shape),
            _full(w_qc), _full(w_qu), _full(w_k), _full(w_v),
            _full(kv_wb), _full(kv_zb), _full(kv_bb),
            _full(ix_wb), _full(ix_zb), _full(ix_bb),
            _full(w_dq), _full(w_iuq), _full(w_w),
            _full(k_scale2),
            _rowtile(cos.shape), _rowtile(sin.shape),
        ],
        out_specs=[
            _rowtile((t, H * DH)),
            _rowtile((t, DH)),
            _rowtile((t, DH + 1)),
            pl.BlockSpec((TQ // M, DH + 1), lambda i: (i, 0)),
            pl.BlockSpec((TQ // M, CI), lambda i: (i, 0)),
            _rowtile((t, NIH * CI)),
            _rowtile((t, NIH)),
        ],
        out_shape=[
            jax.ShapeDtypeStruct((t, H * DH), f32),
            jax.ShapeDtypeStruct((t, DH), f32),
            jax.ShapeDtypeStruct((t, DH + 1), f32),
            jax.ShapeDtypeStruct((nb, DH + 1), f32),
            jax.ShapeDtypeStruct((nb, CI), f32),
            jax.ShapeDtypeStruct((t, NIH * CI), f32),
            jax.ShapeDtypeStruct((t, NIH), f32),
        ],
    )(h2, w_qc, w_qu, w_k, w_v, kv_wb, kv_zb, kv_bb, ix_wb, ix_zb, ix_bb,
      w_dq, w_iuq, w_w, k_scale2, cos, sin)

    import functools
    sel = pl.pallas_call(
        functools.partial(_indexer_kernel, nb=nb),
        grid=(1,),
        in_specs=[
            _full(qi_p), _full(wtok_p), _full(kcomp_i),
        ],
        out_specs=pl.BlockSpec((t, nb), lambda i: (0, 0)),
        out_shape=jax.ShapeDtypeStruct((t, nb), f32),
    )(qi_p, wtok_p, kcomp_i)

    out = pl.pallas_call(
        functools.partial(_attn_kernel, t=t, nb=nb),
        grid=(grid,),
        in_specs=[
            _rowtile(q_raw.shape),
            _full(k_rot), _full(v_aug), _full(kc_aug),
            _rowtile(sel.shape),
            _rowtile(cos.shape), _rowtile(sin.shape),
            _full(q_scale2), _full(sink2),
            _full(gp0), _full(gp1), _full(gp2), _full(gp3), _full(w_final),
        ],
        out_specs=_rowtile((t, D)),
        out_shape=jax.ShapeDtypeStruct((t, D), f32),
    )(q_raw, k_rot, v_aug, kc_aug, sel, cos, sin, q_scale2, sink2,
      gp0, gp1, gp2, gp3, w_final)

    return out.reshape(b, t, D)
